# baseline (speedup 1.0000x reference)
import jax
import jax.numpy as jnp
from jax import lax
from jax.experimental import pallas as pl
from jax.experimental.pallas import tpu as pltpu
from jax.experimental.pallas import tpu_sc as plsc

HE = 10000
D = 128
NP = 10240

_NC = 2
_NS = 16
_TILES = _NC * _NS
_C = 80
_G = _C // 16
_NB = 2


def _bcast_lane(v16, i):
    return jnp.take_along_axis(v16, jnp.full((16,), i, jnp.int32), axis=0)


def _zero_rows(ref, nrows):
    z = jnp.zeros((16,), jnp.float32)

    def body(r, carry):
        for b in range(D // 16):
            ref[r, pl.ds(b * 16, 16)] = z
        return carry

    lax.fori_loop(0, nrows, body, 0)


def _scale_rows_inplace(rows, scal16, g):
    for i in range(16):
        bv = _bcast_lane(scal16, i)
        r = g * 16 + i
        for b in range(D // 16):
            rows[r, pl.ds(b * 16, 16)] = rows[r, pl.ds(b * 16, 16)] * bv


def _zero_acc_slice(acc, rows_buf, s):
    _zero_rows(rows_buf, _C)
    rpt = NP // _NS
    for j in range(rpt // _C):
        r0 = pl.multiple_of(s * rpt + j * _C, 8)
        pltpu.sync_copy(rows_buf, acc.at[pl.ds(r0, _C)])


def _stage_writeback(acc, parts_hbm, rows_buf, c, s):
    rpt = NP // _NS
    for j in range(rpt // _C):
        r0 = pl.multiple_of(s * rpt + j * _C, 8)
        pltpu.sync_copy(acc.at[pl.ds(r0, _C)], rows_buf)
        pltpu.sync_copy(rows_buf, parts_hbm.at[c, pl.ds(r0, _C)])


def _edge_accum_body(nid_hbm, hid_hbm, val_hbm, x_hbm, eparts_hbm,
                     e_acc, nid_v, hid_v, valb0, valb1, rin0, rin1,
                     semg0, semg1, semv0, semv1):
    c = lax.axis_index("c")
    s = lax.axis_index("s")
    wid = c * _NS + s
    per_tile = nid_v.shape[0]
    nch = per_tile // _C
    base = wid * per_tile
    rin = (rin0, rin1)
    valb = (valb0, valb1)
    semg = (semg0, semg1)
    semv = (semv0, semv1)

    b0 = pl.multiple_of(base, 8)
    pltpu.sync_copy(nid_hbm.at[pl.ds(b0, per_tile)], nid_v)
    pltpu.sync_copy(hid_hbm.at[wid], hid_v)

    _zero_acc_slice(e_acc, rin0, s)

    def start_fetch(k, p):
        pltpu.async_copy(x_hbm.at[nid_v.at[pl.ds(k * _C, _C)]], rin[p], semg[p])
        off = pl.multiple_of(base + k * _C, 8)
        pltpu.async_copy(val_hbm.at[pl.ds(off, _C)], valb[p], semv[p])

    def wait_fetch(k, p):
        pltpu.make_async_copy(
            x_hbm.at[nid_v.at[pl.ds(k * _C, _C)]], rin[p], semg[p]).wait()
        off = pl.multiple_of(base + k * _C, 8)
        pltpu.make_async_copy(val_hbm.at[pl.ds(off, _C)], valb[p], semv[p]).wait()

    for p in range(_NB):
        start_fetch(p, p)
    plsc.subcore_barrier()

    def visit(k, p, prefetch, unroll):
        wait_fetch(k, p)
        if unroll:
            for g in range(_G):
                v16 = valb[p][pl.ds(g * 16, 16)]
                _scale_rows_inplace(rin[p], v16, g)
        else:
            def grp(g, carry):
                v16 = valb[p][pl.ds(g * 16, 16)]
                _scale_rows_inplace(rin[p], v16, g)
                return carry

            lax.fori_loop(0, _G, grp, 0)
        pltpu.sync_copy(rin[p], e_acc.at[hid_v.at[k]], add=True)
        if prefetch:
            start_fetch(k + _NB, p)

    def steady(kk, carry):
        visit(kk * _NB, 0, True, True)
        visit(kk * _NB + 1, 1, True, True)
        return carry

    lax.fori_loop(0, (nch - 3) // _NB, steady, 0)
    visit(nch - 3, 0, True, False)
    visit(nch - 2, 1, False, False)
    visit(nch - 1, 0, False, False)

    plsc.subcore_barrier()
    _stage_writeback(e_acc, eparts_hbm, rin0, c, s)


def _attn_accum_body(nid_hbm, hid_hbm, w_hbm, et_hbm,
                     yparts_hbm, dparts_hbm,
                     y_acc, d_acc, nid_v, hid_v,
                     wb0, wb1, rin0, rin1, dbuf,
                     semg0, semg1, semw0, semw1):
    c = lax.axis_index("c")
    s = lax.axis_index("s")
    wid = c * _NS + s
    nch = nid_v.shape[0]
    per_tile = nch * _C
    base = wid * per_tile
    rin = (rin0, rin1)
    wb = (wb0, wb1)
    semg = (semg0, semg1)
    semw = (semw0, semw1)

    b0 = pl.multiple_of(base, 8)
    pltpu.sync_copy(nid_hbm.at[wid], nid_v)
    pltpu.sync_copy(hid_hbm.at[pl.ds(b0, per_tile)], hid_v)

    _zero_acc_slice(y_acc, rin0, s)
    z16 = jnp.zeros((16,), jnp.float32)
    rpt = NP // _NS

    def zd(i, carry):
        dbuf[pl.ds(i * 16, 16)] = z16
        return carry

    lax.fori_loop(0, 8, zd, 0)
    d0 = pl.multiple_of(s * rpt, 8)
    for j in range(rpt // 128):
        pltpu.sync_copy(dbuf, d_acc.at[pl.ds(pl.multiple_of(d0 + j * 128, 8), 128)])

    def start_fetch(k, p):
        hslice = hid_v.at[pl.ds(k * _C, _C)]
        pltpu.async_copy(et_hbm.at[hslice], rin[p], semg[p])
        pltpu.async_copy(w_hbm.at[hslice], wb[p], semw[p])

    def wait_fetch(k, p):
        hslice = hid_v.at[pl.ds(k * _C, _C)]
        pltpu.make_async_copy(et_hbm.at[hslice], rin[p], semg[p]).wait()
        pltpu.make_async_copy(w_hbm.at[hslice], wb[p], semw[p]).wait()

    for p in range(_NB):
        start_fetch(p, p)
    plsc.subcore_barrier()

    def visit(k, p, prefetch):
        wait_fetch(k, p)
        pltpu.sync_copy(wb[p], d_acc.at[nid_v.at[k]], add=True)
        pltpu.sync_copy(rin[p], y_acc.at[nid_v.at[k]], add=True)
        if prefetch:
            start_fetch(k + _NB, p)

    def steady(kk, carry):
        visit(kk * _NB, 0, True)
        visit(kk * _NB + 1, 1, True)
        return carry

    lax.fori_loop(0, (nch - 3) // _NB, steady, 0)
    visit(nch - 3, 0, True)
    visit(nch - 2, 1, False)
    visit(nch - 1, 0, False)

    plsc.subcore_barrier()
    _stage_writeback(y_acc, yparts_hbm, rin0, c, s)
    for j in range(rpt // 128):
        dj = pl.multiple_of(d0 + j * 128, 8)
        pltpu.sync_copy(d_acc.at[pl.ds(dj, 128)], dbuf)
        pltpu.sync_copy(dbuf, dparts_hbm.at[c, pl.ds(dj, 128)])


def _project_body(ep_ref, a2_ref, et_ref, w_ref):
    ep = ep_ref[0] + ep_ref[1]
    t = jnp.sum(ep * a2_ref[...], axis=1, keepdims=True)
    w = jnp.exp(t)
    w_ref[...] = w
    et_ref[...] = ep * w


def _finalize_body(yp_ref, dp_ref, out_ref):
    y = yp_ref[0] + yp_ref[1]
    dsum = dp_ref[0] + dp_ref[1]
    out_ref[...] = y / (dsum + 1e-16)


def kernel(H_indices, H_values, X, a):
    n_nodes, d = X.shape
    nnz = H_values.shape[0]
    per_tile = nnz // _TILES
    nch = per_tile // _C
    nid_flat = H_indices[0].astype(jnp.int32)
    hid_flat = H_indices[1].astype(jnp.int32)
    nid3 = nid_flat.reshape(_TILES, nch, _C)
    hid3 = hid_flat.reshape(_TILES, nch, _C)
    vals = H_values.astype(jnp.float32)

    mesh = plsc.VectorSubcoreMesh(core_axis_name="c", subcore_axis_name="s")
    sc_params = pltpu.CompilerParams(needs_layout_passes=False)

    edge_accum = pl.kernel(
        _edge_accum_body,
        out_type=jax.ShapeDtypeStruct((_NC, NP, D), jnp.float32),
        mesh=mesh,
        compiler_params=sc_params,
        scratch_types=[
            pltpu.VMEM_SHARED((NP, D), jnp.float32),
            pltpu.VMEM((per_tile,), jnp.int32),
            pltpu.VMEM((nch, _C), jnp.int32),
            pltpu.VMEM((_C,), jnp.float32),
            pltpu.VMEM((_C,), jnp.float32),
            pltpu.VMEM((_C, D), jnp.float32),
            pltpu.VMEM((_C, D), jnp.float32),
            pltpu.SemaphoreType.DMA,
            pltpu.SemaphoreType.DMA,
            pltpu.SemaphoreType.DMA,
            pltpu.SemaphoreType.DMA,
        ],
    )
    e_parts = edge_accum(nid_flat, hid3, vals, X)

    r_blk = 1024
    et_full, w2 = pl.pallas_call(
        _project_body,
        grid=(NP // r_blk,),
        in_specs=[
            pl.BlockSpec((_NC, r_blk, D), lambda i: (0, i, 0)),
            pl.BlockSpec((1, D), lambda i: (0, 0)),
        ],
        out_specs=[
            pl.BlockSpec((r_blk, D), lambda i: (i, 0)),
            pl.BlockSpec((r_blk, 1), lambda i: (i, 0)),
        ],
        out_shape=[
            jax.ShapeDtypeStruct((NP, D), jnp.float32),
            jax.ShapeDtypeStruct((NP, 1), jnp.float32),
        ],
    )(e_parts, a[d:].reshape(1, d))
    w_tab = w2.reshape(-1)

    attn_accum = pl.kernel(
        _attn_accum_body,
        out_type=[
            jax.ShapeDtypeStruct((_NC, NP, D), jnp.float32),
            jax.ShapeDtypeStruct((_NC, NP), jnp.float32),
        ],
        mesh=mesh,
        compiler_params=sc_params,
        scratch_types=[
            pltpu.VMEM_SHARED((NP, D), jnp.float32),
            pltpu.VMEM_SHARED((NP,), jnp.float32),
            pltpu.VMEM((nch, _C), jnp.int32),
            pltpu.VMEM((per_tile,), jnp.int32),
            pltpu.VMEM((_C,), jnp.float32),
            pltpu.VMEM((_C,), jnp.float32),
            pltpu.VMEM((_C, D), jnp.float32),
            pltpu.VMEM((_C, D), jnp.float32),
            pltpu.VMEM((128,), jnp.float32),
            pltpu.SemaphoreType.DMA,
            pltpu.SemaphoreType.DMA,
            pltpu.SemaphoreType.DMA,
            pltpu.SemaphoreType.DMA,
        ],
    )
    y_parts, d_parts = attn_accum(nid3, hid_flat, w_tab, et_full)

    dp3 = d_parts.reshape(_NC, NP, 1)
    out_pad = pl.pallas_call(
        _finalize_body,
        grid=(NP // r_blk,),
        in_specs=[
            pl.BlockSpec((_NC, r_blk, D), lambda i: (0, i, 0)),
            pl.BlockSpec((_NC, r_blk, 1), lambda i: (0, i, 0)),
        ],
        out_specs=pl.BlockSpec((r_blk, D), lambda i: (i, 0)),
        out_shape=jax.ShapeDtypeStruct((NP, D), jnp.float32),
    )(y_parts, dp3)
    return out_pad[:n_nodes]


# speedup vs baseline: 1.0156x; 1.0156x over previous
import jax
import jax.numpy as jnp
from jax import lax
from jax.experimental import pallas as pl
from jax.experimental.pallas import tpu as pltpu
from jax.experimental.pallas import tpu_sc as plsc

HE = 10000
D = 128
NP = 10240

_NC = 2
_NS = 16
_TILES = _NC * _NS
_C = 80
_G = _C // 16
_NB = 2


def _bcast_lane(v16, i):
    return jnp.take_along_axis(v16, jnp.full((16,), i, jnp.int32), axis=0)


def _zero_rows(ref, nrows):
    z = jnp.zeros((16,), jnp.float32)

    def body(r, carry):
        for b in range(D // 16):
            ref[r, pl.ds(b * 16, 16)] = z
        return carry

    lax.fori_loop(0, nrows, body, 0)


def _scale_rows(src, dst, scal16, g):
    for i in range(16):
        bv = _bcast_lane(scal16, i)
        r = g * 16 + i
        for b in range(D // 16):
            dst[r, pl.ds(b * 16, 16)] = src[r, pl.ds(b * 16, 16)] * bv


def _zero_acc_slice(acc, rows_buf, s):
    _zero_rows(rows_buf, _C)
    rpt = NP // _NS
    for j in range(rpt // _C):
        r0 = pl.multiple_of(s * rpt + j * _C, 8)
        pltpu.sync_copy(rows_buf, acc.at[pl.ds(r0, _C)])


def _stage_writeback(acc, parts_hbm, rows_buf, c, s):
    rpt = NP // _NS
    for j in range(rpt // _C):
        r0 = pl.multiple_of(s * rpt + j * _C, 8)
        pltpu.sync_copy(acc.at[pl.ds(r0, _C)], rows_buf)
        pltpu.sync_copy(rows_buf, parts_hbm.at[c, pl.ds(r0, _C)])


def _edge_accum_body(nid_hbm, hid_hbm, val_hbm, x_hbm, eparts_hbm,
                     e_acc, hid_v, nidb0, nidb1, valb0, valb1,
                     rin0, rin1, rout,
                     semg0, semg1, semv0, semv1, semn0, semn1):
    c = lax.axis_index("c")
    s = lax.axis_index("s")
    wid = c * _NS + s
    nch = hid_v.shape[0]
    per_tile = nch * _C
    base = wid * per_tile
    rin = (rin0, rin1)
    nidb = (nidb0, nidb1)
    valb = (valb0, valb1)
    semg = (semg0, semg1)
    semv = (semv0, semv1)
    semn = (semn0, semn1)

    pltpu.sync_copy(hid_hbm.at[wid], hid_v)

    def start_idx(k, p):
        off = pl.multiple_of(base + k * _C, 8)
        pltpu.async_copy(nid_hbm.at[pl.ds(off, _C)], nidb[p], semn[p])
        pltpu.async_copy(val_hbm.at[pl.ds(off, _C)], valb[p], semv[p])

    def start_gather(p):
        pltpu.async_copy(x_hbm.at[nidb[p]], rin[p], semg[p])

    for p in range(_NB):
        start_idx(p, p)
    for p in range(_NB):
        pltpu.make_async_copy(nid_hbm.at[pl.ds(0, _C)], nidb[p], semn[p]).wait()
        start_gather(p)

    _zero_acc_slice(e_acc, rout, s)
    plsc.subcore_barrier()

    def visit(k, p, prefetch):
        pltpu.make_async_copy(x_hbm.at[nidb[p]], rin[p], semg[p]).wait()
        pltpu.make_async_copy(val_hbm.at[pl.ds(0, _C)], valb[p], semv[p]).wait()
        if prefetch:
            off = pl.multiple_of(base + (k + _NB) * _C, 8)
            pltpu.async_copy(nid_hbm.at[pl.ds(off, _C)], nidb[p], semn[p])
        for g in range(_G):
            v16 = valb[p][pl.ds(g * 16, 16)]
            _scale_rows(rin[p], rout, v16, g)
        if prefetch:
            pltpu.make_async_copy(
                nid_hbm.at[pl.ds(0, _C)], nidb[p], semn[p]).wait()
            start_gather(p)
            off = pl.multiple_of(base + (k + _NB) * _C, 8)
            pltpu.async_copy(val_hbm.at[pl.ds(off, _C)], valb[p], semv[p])
        pltpu.sync_copy(rout, e_acc.at[hid_v.at[k]], add=True)

    def steady(kk, carry):
        visit(kk * _NB, 0, True)
        visit(kk * _NB + 1, 1, True)
        return carry

    lax.fori_loop(0, (nch - 3) // _NB, steady, 0)
    visit(nch - 3, 0, True)
    visit(nch - 2, 1, False)
    visit(nch - 1, 0, False)

    plsc.subcore_barrier()
    _stage_writeback(e_acc, eparts_hbm, rin0, c, s)


def _attn_accum_body(nid_hbm, hid_hbm, w_hbm, et_hbm,
                     yparts_hbm, dparts_hbm,
                     y_acc, d_acc, nid_v, hid_v,
                     wb0, wb1, rin0, rin1, dbuf,
                     semg0, semg1, semw0, semw1):
    c = lax.axis_index("c")
    s = lax.axis_index("s")
    wid = c * _NS + s
    nch = nid_v.shape[0]
    per_tile = nch * _C
    base = wid * per_tile
    rin = (rin0, rin1)
    wb = (wb0, wb1)
    semg = (semg0, semg1)
    semw = (semw0, semw1)

    b0 = pl.multiple_of(base, 8)
    pltpu.sync_copy(nid_hbm.at[wid], nid_v)
    pltpu.sync_copy(hid_hbm.at[pl.ds(b0, per_tile)], hid_v)

    _zero_acc_slice(y_acc, rin0, s)
    z16 = jnp.zeros((16,), jnp.float32)
    rpt = NP // _NS

    def zd(i, carry):
        dbuf[pl.ds(i * 16, 16)] = z16
        return carry

    lax.fori_loop(0, 8, zd, 0)
    d0 = pl.multiple_of(s * rpt, 8)
    for j in range(rpt // 128):
        pltpu.sync_copy(dbuf, d_acc.at[pl.ds(pl.multiple_of(d0 + j * 128, 8), 128)])

    def start_fetch(k, p):
        hslice = hid_v.at[pl.ds(k * _C, _C)]
        pltpu.async_copy(et_hbm.at[hslice], rin[p], semg[p])
        pltpu.async_copy(w_hbm.at[hslice], wb[p], semw[p])

    def wait_fetch(k, p):
        hslice = hid_v.at[pl.ds(k * _C, _C)]
        pltpu.make_async_copy(et_hbm.at[hslice], rin[p], semg[p]).wait()
        pltpu.make_async_copy(w_hbm.at[hslice], wb[p], semw[p]).wait()

    for p in range(_NB):
        start_fetch(p, p)
    plsc.subcore_barrier()

    def visit(k, p, prefetch):
        wait_fetch(k, p)
        pltpu.sync_copy(wb[p], d_acc.at[nid_v.at[k]], add=True)
        pltpu.sync_copy(rin[p], y_acc.at[nid_v.at[k]], add=True)
        if prefetch:
            start_fetch(k + _NB, p)

    def steady(kk, carry):
        visit(kk * _NB, 0, True)
        visit(kk * _NB + 1, 1, True)
        return carry

    lax.fori_loop(0, (nch - 3) // _NB, steady, 0)
    visit(nch - 3, 0, True)
    visit(nch - 2, 1, False)
    visit(nch - 1, 0, False)

    plsc.subcore_barrier()
    _stage_writeback(y_acc, yparts_hbm, rin0, c, s)
    for j in range(rpt // 128):
        dj = pl.multiple_of(d0 + j * 128, 8)
        pltpu.sync_copy(d_acc.at[pl.ds(dj, 128)], dbuf)
        pltpu.sync_copy(dbuf, dparts_hbm.at[c, pl.ds(dj, 128)])


def _project_body(ep_ref, a2_ref, et_ref, w_ref):
    ep = ep_ref[0] + ep_ref[1]
    t = jnp.sum(ep * a2_ref[...], axis=1, keepdims=True)
    w = jnp.exp(t)
    w_ref[...] = w
    et_ref[...] = ep * w


def _finalize_body(yp_ref, dp_ref, out_ref):
    y = yp_ref[0] + yp_ref[1]
    dsum = dp_ref[0] + dp_ref[1]
    out_ref[...] = y / (dsum + 1e-16)


def kernel(H_indices, H_values, X, a):
    n_nodes, d = X.shape
    nnz = H_values.shape[0]
    per_tile = nnz // _TILES
    nch = per_tile // _C
    nid_flat = H_indices[0].astype(jnp.int32)
    hid_flat = H_indices[1].astype(jnp.int32)
    nid3 = nid_flat.reshape(_TILES, nch, _C)
    hid3 = hid_flat.reshape(_TILES, nch, _C)
    vals = H_values.astype(jnp.float32)

    mesh = plsc.VectorSubcoreMesh(core_axis_name="c", subcore_axis_name="s")
    sc_params = pltpu.CompilerParams(needs_layout_passes=False)

    edge_accum = pl.kernel(
        _edge_accum_body,
        out_type=jax.ShapeDtypeStruct((_NC, NP, D), jnp.float32),
        mesh=mesh,
        compiler_params=sc_params,
        scratch_types=[
            pltpu.VMEM_SHARED((NP, D), jnp.float32),
            pltpu.VMEM((nch, _C), jnp.int32),
            pltpu.VMEM((_C,), jnp.int32),
            pltpu.VMEM((_C,), jnp.int32),
            pltpu.VMEM((_C,), jnp.float32),
            pltpu.VMEM((_C,), jnp.float32),
            pltpu.VMEM((_C, D), jnp.float32),
            pltpu.VMEM((_C, D), jnp.float32),
            pltpu.VMEM((_C, D), jnp.float32),
            pltpu.SemaphoreType.DMA,
            pltpu.SemaphoreType.DMA,
            pltpu.SemaphoreType.DMA,
            pltpu.SemaphoreType.DMA,
            pltpu.SemaphoreType.DMA,
            pltpu.SemaphoreType.DMA,
        ],
    )
    e_parts = edge_accum(nid_flat, hid3, vals, X)

    r_blk = 1024
    et_full, w2 = pl.pallas_call(
        _project_body,
        grid=(NP // r_blk,),
        in_specs=[
            pl.BlockSpec((_NC, r_blk, D), lambda i: (0, i, 0)),
            pl.BlockSpec((1, D), lambda i: (0, 0)),
        ],
        out_specs=[
            pl.BlockSpec((r_blk, D), lambda i: (i, 0)),
            pl.BlockSpec((r_blk, 1), lambda i: (i, 0)),
        ],
        out_shape=[
            jax.ShapeDtypeStruct((NP, D), jnp.float32),
            jax.ShapeDtypeStruct((NP, 1), jnp.float32),
        ],
    )(e_parts, a[d:].reshape(1, d))
    w_tab = w2.reshape(-1)

    attn_accum = pl.kernel(
        _attn_accum_body,
        out_type=[
            jax.ShapeDtypeStruct((_NC, NP, D), jnp.float32),
            jax.ShapeDtypeStruct((_NC, NP), jnp.float32),
        ],
        mesh=mesh,
        compiler_params=sc_params,
        scratch_types=[
            pltpu.VMEM_SHARED((NP, D), jnp.float32),
            pltpu.VMEM_SHARED((NP,), jnp.float32),
            pltpu.VMEM((nch, _C), jnp.int32),
            pltpu.VMEM((per_tile,), jnp.int32),
            pltpu.VMEM((_C,), jnp.float32),
            pltpu.VMEM((_C,), jnp.float32),
            pltpu.VMEM((_C, D), jnp.float32),
            pltpu.VMEM((_C, D), jnp.float32),
            pltpu.VMEM((128,), jnp.float32),
            pltpu.SemaphoreType.DMA,
            pltpu.SemaphoreType.DMA,
            pltpu.SemaphoreType.DMA,
            pltpu.SemaphoreType.DMA,
        ],
    )
    y_parts, d_parts = attn_accum(nid3, hid_flat, w_tab, et_full)

    dp3 = d_parts.reshape(_NC, NP, 1)
    out_pad = pl.pallas_call(
        _finalize_body,
        grid=(NP // r_blk,),
        in_specs=[
            pl.BlockSpec((_NC, r_blk, D), lambda i: (0, i, 0)),
            pl.BlockSpec((_NC, r_blk, 1), lambda i: (0, i, 0)),
        ],
        out_specs=pl.BlockSpec((r_blk, D), lambda i: (i, 0)),
        out_shape=jax.ShapeDtypeStruct((NP, D), jnp.float32),
    )(y_parts, dp3)
    return out_pad[:n_nodes]
